# Initial kernel scaffold; baseline (speedup 1.0000x reference)
import jax
import jax.numpy as jnp
from jax.experimental import pallas as pl


def kernel(word_vector, context_vector, table):
    raise NotImplementedError("write your pallas kernel here")


# speedup vs baseline: 5.6534x; 5.6534x over previous
import functools

import jax
import jax.numpy as jnp
from jax import lax
from jax.experimental import pallas as pl
from jax.experimental.pallas import tpu as pltpu
from jax.experimental.pallas import tpu_sc as plsc

NC = 2
NS = 16
L = 16
NW = NC * NS

NB = 32
IDXCHUNK = 128


def _make_kernel(B, K, D, V):
    bpw = B // NW
    nchunk = bpw // NB
    nidx = (NB * K) // IDXCHUNK
    assert NB * K == nidx * IDXCHUNK
    assert D % L == 0

    mesh = plsc.VectorSubcoreMesh(core_axis_name="c", subcore_axis_name="s")

    @functools.partial(
        pl.kernel,
        mesh=mesh,
        compiler_params=pltpu.CompilerParams(
            needs_layout_passes=False, use_tc_tiling_on_sc=False),
        out_type=jax.ShapeDtypeStruct((B, K), jnp.float32),
        scratch_types=[
            pltpu.VMEM((bpw,), jnp.int32),
            pltpu.VMEM((bpw * K,), jnp.int32),
            pltpu.VMEM((NB, D), jnp.float32),
            pltpu.VMEM((NB * K, D), jnp.float32),
            pltpu.VMEM((NB, K), jnp.float32),
            pltpu.SemaphoreType.DMA,
        ],
    )
    def k(table_hbm, widx_hbm, cidx_hbm, out_hbm,
          widx_v, cidx_v, wrows_v, crows_v, out_v, sem):
        wid = lax.axis_index("s") * NC + lax.axis_index("c")
        base_b = wid * bpw
        pltpu.sync_copy(widx_hbm.at[pl.ds(base_b, bpw)], widx_v)
        pltpu.sync_copy(cidx_hbm.at[pl.ds(base_b * K, bpw * K)], cidx_v)

        iota = lax.iota(jnp.int32, 16)

        def chunk_body(c, carry):
            woff = pl.multiple_of(c * NB, NB)
            coff = pl.multiple_of(c * (NB * K), NB * K)
            cw = pltpu.async_copy(
                table_hbm.at[widx_v.at[pl.ds(woff, NB)]], wrows_v, sem)
            ccs = [
                pltpu.async_copy(
                    table_hbm.at[cidx_v.at[pl.ds(coff + j * IDXCHUNK, IDXCHUNK)]],
                    crows_v.at[pl.ds(j * IDXCHUNK, IDXCHUNK)], sem)
                for j in range(nidx)
            ]
            cw.wait()
            for cc in ccs:
                cc.wait()

            for g in range(NB // L):
                rows16 = iota + (g * L)
                crow_base = [rows16 * K + kk for kk in range(K)]

                def dbody(d, accs):
                    dcol = jnp.full((L,), 0, jnp.int32) + d
                    wcol = plsc.load_gather(wrows_v, [rows16, dcol])
                    return tuple(
                        accs[kk] + wcol * plsc.load_gather(
                            crows_v, [crow_base[kk], dcol])
                        for kk in range(K))

                accs = lax.fori_loop(
                    0, D, dbody,
                    tuple(jnp.zeros((L,), jnp.float32) for _ in range(K)))

                for kk in range(K):
                    y = 1.0 / (1.0 + jnp.exp(-accs[kk]))
                    plsc.store_scatter(
                        out_v, [rows16, jnp.full((L,), kk, jnp.int32)], y)

            pltpu.sync_copy(out_v, out_hbm.at[pl.ds(base_b + woff, NB)])
            return carry

        lax.fori_loop(0, nchunk, chunk_body, 0)

    return k


def kernel(word_vector, context_vector, table):
    B, K = context_vector.shape
    V, D = table.shape
    widx = word_vector.reshape(-1)
    cidx = context_vector.reshape(-1)
    k = _make_kernel(B, K, D, V)
    return k(table, widx, cidx)


# speedup vs baseline: 5.7485x; 1.0168x over previous
import functools

import jax
import jax.numpy as jnp
from jax import lax
from jax.experimental import pallas as pl
from jax.experimental.pallas import tpu as pltpu
from jax.experimental.pallas import tpu_sc as plsc

NC = 2
NS = 16
L = 16
NW = NC * NS

NB = 32
IDXCHUNK = 128
UNROLL = 8


def _make_kernel(B, K, D, V):
    bpw = B // NW
    nchunk = bpw // NB
    nidx = (NB * K) // IDXCHUNK
    assert NB * K == nidx * IDXCHUNK
    assert D % L == 0 and nchunk % 2 == 0

    mesh = plsc.VectorSubcoreMesh(core_axis_name="c", subcore_axis_name="s")

    @functools.partial(
        pl.kernel,
        mesh=mesh,
        compiler_params=pltpu.CompilerParams(
            needs_layout_passes=False, use_tc_tiling_on_sc=False),
        out_type=jax.ShapeDtypeStruct((B, K), jnp.float32),
        scratch_types=[
            pltpu.VMEM((bpw,), jnp.int32),
            pltpu.VMEM((bpw * K,), jnp.int32),
            pltpu.VMEM((NB, D), jnp.float32),
            pltpu.VMEM((NB, D), jnp.float32),
            pltpu.VMEM((NB * K, D), jnp.float32),
            pltpu.VMEM((NB * K, D), jnp.float32),
            pltpu.VMEM((NB, K), jnp.float32),
            pltpu.VMEM((NB, K), jnp.float32),
            pltpu.SemaphoreType.DMA,
            pltpu.SemaphoreType.DMA,
        ],
    )
    def k(table_hbm, widx_hbm, cidx_hbm, out_hbm,
          widx_v, cidx_v, wrows0, wrows1, crows0, crows1, out0, out1,
          sem0, sem1):
        wid = lax.axis_index("s") * NC + lax.axis_index("c")
        base_b = wid * bpw
        pltpu.sync_copy(widx_hbm.at[pl.ds(base_b, bpw)], widx_v)
        pltpu.sync_copy(cidx_hbm.at[pl.ds(base_b * K, bpw * K)], cidx_v)

        wrows = (wrows0, wrows1)
        crows = (crows0, crows1)
        outs = (out0, out1)
        sems = (sem0, sem1)
        iota = lax.iota(jnp.int32, 16)

        def fire(c, buf):
            woff = pl.multiple_of(c * NB, NB)
            coff = pl.multiple_of(c * (NB * K), NB * K)
            pltpu.async_copy(
                table_hbm.at[widx_v.at[pl.ds(woff, NB)]], wrows[buf],
                sems[buf])
            for j in range(nidx):
                pltpu.async_copy(
                    table_hbm.at[cidx_v.at[pl.ds(coff + j * IDXCHUNK,
                                                 IDXCHUNK)]],
                    crows[buf].at[pl.ds(j * IDXCHUNK, IDXCHUNK)], sems[buf])

        def drain(buf):
            pltpu.make_async_copy(
                table_hbm.at[pl.ds(0, NB)], wrows[buf], sems[buf]).wait()
            pltpu.make_async_copy(
                table_hbm.at[pl.ds(0, NB * K)], crows[buf], sems[buf]).wait()

        def compute(c, buf):
            for g in range(NB // L):
                rows16 = iota + (g * L)
                crow_base = [rows16 * K + kk for kk in range(K)]

                def dbody(d, accs):
                    dcol = jnp.full((L,), 0, jnp.int32) + d
                    wcol = plsc.load_gather(wrows[buf], [rows16, dcol])
                    return tuple(
                        accs[kk] + wcol * plsc.load_gather(
                            crows[buf], [crow_base[kk], dcol])
                        for kk in range(K))

                accs = lax.fori_loop(
                    0, D, dbody,
                    tuple(jnp.zeros((L,), jnp.float32) for _ in range(K)),
                    unroll=UNROLL)

                for kk in range(K):
                    y = 1.0 / (1.0 + jnp.exp(-accs[kk]))
                    plsc.store_scatter(
                        outs[buf], [rows16, jnp.full((L,), kk, jnp.int32)], y)

            woff = pl.multiple_of(c * NB, NB)
            pltpu.sync_copy(outs[buf], out_hbm.at[pl.ds(base_b + woff, NB)])

        fire(0, 0)

        def pair_body(p, carry):
            c0 = p * 2
            fire(c0 + 1, 1)
            drain(0)
            compute(c0, 0)
            fire(jnp.minimum(c0 + 2, nchunk - 2), 0)
            drain(1)
            compute(c0 + 1, 1)
            return carry

        lax.fori_loop(0, nchunk // 2, pair_body, 0)
        drain(0)

    return k


def kernel(word_vector, context_vector, table):
    B, K = context_vector.shape
    V, D = table.shape
    widx = word_vector.reshape(-1)
    cidx = context_vector.reshape(-1)
    k = _make_kernel(B, K, D, V)
    return k(table, widx, cidx)


# speedup vs baseline: 7.8485x; 1.3653x over previous
import functools

import jax
import jax.numpy as jnp
from jax import lax
from jax.experimental import pallas as pl
from jax.experimental.pallas import tpu as pltpu
from jax.experimental.pallas import tpu_sc as plsc

NC = 2
NS = 16
L = 16
NW = NC * NS

NB = 32
IDXCHUNK = 128
UNROLL = 8


def _make_kernel(B, K, D, V):
    bpw = B // NW
    nchunk = bpw // NB
    nidx = (NB * K) // IDXCHUNK
    assert NB * K == nidx * IDXCHUNK
    assert D % L == 0 and nchunk % 2 == 0

    mesh = plsc.VectorSubcoreMesh(core_axis_name="c", subcore_axis_name="s")

    @functools.partial(
        pl.kernel,
        mesh=mesh,
        compiler_params=pltpu.CompilerParams(
            needs_layout_passes=False, use_tc_tiling_on_sc=False),
        out_type=jax.ShapeDtypeStruct((B, K), jnp.float32),
        scratch_types=[
            pltpu.VMEM((bpw,), jnp.int32),
            pltpu.VMEM((bpw * K,), jnp.int32),
            pltpu.VMEM((NB, D), jnp.float32),
            pltpu.VMEM((NB, D), jnp.float32),
            pltpu.VMEM((NB * K, D), jnp.float32),
            pltpu.VMEM((NB * K, D), jnp.float32),
            pltpu.VMEM((NB, K), jnp.float32),
            pltpu.VMEM((NB, K), jnp.float32),
            pltpu.SemaphoreType.DMA,
            pltpu.SemaphoreType.DMA,
        ],
    )
    def k(table_hbm, widx_hbm, cidx_hbm, out_hbm,
          widx_v, cidx_v, wrows0, wrows1, crows0, crows1, out0, out1,
          sem0, sem1):
        wid = lax.axis_index("s") * NC + lax.axis_index("c")
        base_b = wid * bpw
        pltpu.sync_copy(widx_hbm.at[pl.ds(base_b, bpw)], widx_v)
        pltpu.sync_copy(cidx_hbm.at[pl.ds(base_b * K, bpw * K)], cidx_v)

        wrows = (wrows0, wrows1)
        crows = (crows0, crows1)
        outs = (out0, out1)
        sems = (sem0, sem1)
        iota = lax.iota(jnp.int32, 16)

        def fire(c, buf):
            woff = pl.multiple_of(c * NB, NB)
            coff = pl.multiple_of(c * (NB * K), NB * K)
            pltpu.async_copy(
                table_hbm.at[widx_v.at[pl.ds(woff, NB)]], wrows[buf],
                sems[buf])
            for j in range(nidx):
                pltpu.async_copy(
                    table_hbm.at[cidx_v.at[pl.ds(coff + j * IDXCHUNK,
                                                 IDXCHUNK)]],
                    crows[buf].at[pl.ds(j * IDXCHUNK, IDXCHUNK)], sems[buf])

        def drain(buf):
            pltpu.make_async_copy(
                table_hbm.at[pl.ds(0, NB)], wrows[buf], sems[buf]).wait()
            pltpu.make_async_copy(
                table_hbm.at[pl.ds(0, NB * K)], crows[buf], sems[buf]).wait()

        def compute(c, buf):
            for g in range(NB // L):
                rows16 = iota + (g * L)
                crow_base = [rows16 * K + kk for kk in range(K)]

                def dbody(d, accs):
                    dcol = (iota + d) & (D - 1)
                    wcol = plsc.load_gather(wrows[buf], [rows16, dcol])
                    return tuple(
                        accs[kk] + wcol * plsc.load_gather(
                            crows[buf], [crow_base[kk], dcol])
                        for kk in range(K))

                accs = lax.fori_loop(
                    0, D, dbody,
                    tuple(jnp.zeros((L,), jnp.float32) for _ in range(K)),
                    unroll=UNROLL)

                for kk in range(K):
                    y = 1.0 / (1.0 + jnp.exp(-accs[kk]))
                    plsc.store_scatter(
                        outs[buf], [rows16, jnp.full((L,), kk, jnp.int32)], y)

            woff = pl.multiple_of(c * NB, NB)
            pltpu.sync_copy(outs[buf], out_hbm.at[pl.ds(base_b + woff, NB)])

        fire(0, 0)

        def pair_body(p, carry):
            c0 = p * 2
            fire(c0 + 1, 1)
            drain(0)
            compute(c0, 0)
            fire(jnp.minimum(c0 + 2, nchunk - 2), 0)
            drain(1)
            compute(c0 + 1, 1)
            return carry

        lax.fori_loop(0, nchunk // 2, pair_body, 0)
        drain(0)

    return k


def kernel(word_vector, context_vector, table):
    B, K = context_vector.shape
    V, D = table.shape
    widx = word_vector.reshape(-1)
    cidx = context_vector.reshape(-1)
    k = _make_kernel(B, K, D, V)
    return k(table, widx, cidx)


# speedup vs baseline: 12.7999x; 1.6309x over previous
import functools

import jax
import jax.numpy as jnp
from jax import lax
from jax.experimental import pallas as pl
from jax.experimental.pallas import tpu as pltpu
from jax.experimental.pallas import tpu_sc as plsc

NC = 2
NS = 16
L = 16
NW = NC * NS

NB = 16
IDXSTREAM = 80
UNROLL = 8
DPAD = 128


def _make_kernel(B, K, D, V):
    bpw = B // NW
    nchunk = bpw // NB
    nidx = (NB * K) // IDXSTREAM
    assert NB * K == nidx * IDXSTREAM and IDXSTREAM % 8 == 0
    assert D % L == 0 and nchunk % 2 == 0 and NB == L

    mesh = plsc.VectorSubcoreMesh(core_axis_name="c", subcore_axis_name="s")

    @functools.partial(
        pl.kernel,
        mesh=mesh,
        compiler_params=pltpu.CompilerParams(
            needs_layout_passes=False, use_tc_tiling_on_sc=True),
        out_type=jax.ShapeDtypeStruct((B, K), jnp.float32),
        scratch_types=[
            pltpu.VMEM((bpw,), jnp.int32),
            pltpu.VMEM((bpw * K,), jnp.int32),
            pltpu.VMEM((NB, DPAD), jnp.float32),
            pltpu.VMEM((NB, DPAD), jnp.float32),
            pltpu.VMEM((NB * K, DPAD), jnp.float32),
            pltpu.VMEM((NB * K, DPAD), jnp.float32),
            pltpu.VMEM((NB, K), jnp.float32),
            pltpu.VMEM((NB, K), jnp.float32),
            pltpu.SemaphoreType.DMA,
            pltpu.SemaphoreType.DMA,
        ],
    )
    def k(table_hbm, widx_hbm, cidx_hbm, out_hbm,
          widx_v, cidx_v, wrows0, wrows1, crows0, crows1, out0, out1,
          sem0, sem1):
        wid = lax.axis_index("s") * NC + lax.axis_index("c")
        base_b = wid * bpw
        pltpu.sync_copy(widx_hbm.at[pl.ds(base_b, bpw)], widx_v)
        pltpu.sync_copy(cidx_hbm.at[pl.ds(base_b * K, bpw * K)], cidx_v)

        wrows = (wrows0, wrows1)
        crows = (crows0, crows1)
        outs = (out0, out1)
        sems = (sem0, sem1)
        iota = lax.iota(jnp.int32, 16)

        def fire(c, buf):
            woff = pl.multiple_of(c * NB, NB)
            coff = pl.multiple_of(c * (NB * K), NB * K)
            pltpu.async_copy(
                table_hbm.at[widx_v.at[pl.ds(woff, NB)]], wrows[buf],
                sems[buf])
            for j in range(nidx):
                pltpu.async_copy(
                    table_hbm.at[cidx_v.at[pl.ds(coff + j * IDXSTREAM,
                                                 IDXSTREAM)]],
                    crows[buf].at[pl.ds(j * IDXSTREAM, IDXSTREAM)], sems[buf])

        def drain(buf):
            pltpu.make_async_copy(
                table_hbm.at[pl.ds(0, NB)], wrows[buf], sems[buf]).wait()
            pltpu.make_async_copy(
                table_hbm.at[pl.ds(0, NB * K)], crows[buf], sems[buf]).wait()

        def compute(c, buf):
            crow_base = [iota * K + kk for kk in range(K)]

            def dbody(d, accs):
                dcol = (iota + d) & (D - 1)
                wcol = plsc.load_gather(wrows[buf], [iota, dcol])
                return tuple(
                    accs[kk] + wcol * plsc.load_gather(
                        crows[buf], [crow_base[kk], dcol])
                    for kk in range(K))

            accs = lax.fori_loop(
                0, D, dbody,
                tuple(jnp.zeros((L,), jnp.float32) for _ in range(K)),
                unroll=UNROLL)

            for kk in range(K):
                y = 1.0 / (1.0 + jnp.exp(-accs[kk]))
                plsc.store_scatter(
                    outs[buf], [iota, jnp.full((L,), kk, jnp.int32)], y)

            woff = pl.multiple_of(c * NB, NB)
            pltpu.sync_copy(outs[buf], out_hbm.at[pl.ds(base_b + woff, NB)])

        fire(0, 0)

        def pair_body(p, carry):
            c0 = p * 2
            fire(c0 + 1, 1)
            drain(0)
            compute(c0, 0)
            fire(jnp.minimum(c0 + 2, nchunk - 2), 0)
            drain(1)
            compute(c0 + 1, 1)
            return carry

        lax.fori_loop(0, nchunk // 2, pair_body, 0)
        drain(0)

    return k


_TBLK = 8192


def _transpose_pad(table):
    V, D = table.shape
    tab_t = table.T
    grid = pl.cdiv(V, _TBLK)
    eye = jnp.eye(D, DPAD, dtype=jnp.float32)

    def body(tt_ref, eye_ref, out_ref):
        out_ref[...] = jax.lax.dot_general(
            tt_ref[...], eye_ref[...], (((0,), (0,)), ((), ())),
            preferred_element_type=jnp.float32)

    return pl.pallas_call(
        body,
        grid=(grid,),
        in_specs=[
            pl.BlockSpec((D, _TBLK), lambda i: (0, i)),
            pl.BlockSpec((D, DPAD), lambda i: (0, 0)),
        ],
        out_specs=pl.BlockSpec((_TBLK, DPAD), lambda i: (i, 0)),
        out_shape=jax.ShapeDtypeStruct((V, DPAD), jnp.float32),
        compiler_params=pltpu.CompilerParams(
            dimension_semantics=("arbitrary",)),
    )(tab_t, eye)


def kernel(word_vector, context_vector, table):
    B, K = context_vector.shape
    V, D = table.shape
    widx = word_vector.reshape(-1)
    cidx = context_vector.reshape(-1)
    table_pad = _transpose_pad(table)
    k = _make_kernel(B, K, D, V)
    return k(table_pad, widx, cidx)


# speedup vs baseline: 13.4584x; 1.0514x over previous
import functools

import jax
import jax.numpy as jnp
from jax import lax
from jax.experimental import pallas as pl
from jax.experimental.pallas import tpu as pltpu
from jax.experimental.pallas import tpu_sc as plsc

NC = 2
NS = 16
L = 16
NW = NC * NS

NB = 16
IDXSTREAM = 80
UNROLL = 8
DPAD = 128


def _make_kernel(B, K, D, V):
    bpw = B // NW
    nchunk = bpw // NB
    nidx = (NB * K) // IDXSTREAM
    assert NB * K == nidx * IDXSTREAM and IDXSTREAM % 8 == 0
    assert D % L == 0 and nchunk % 2 == 0 and NB == L

    mesh = plsc.VectorSubcoreMesh(core_axis_name="c", subcore_axis_name="s")

    @functools.partial(
        pl.kernel,
        mesh=mesh,
        compiler_params=pltpu.CompilerParams(
            needs_layout_passes=False, use_tc_tiling_on_sc=True),
        out_type=jax.ShapeDtypeStruct((B, K), jnp.float32),
        scratch_types=[
            pltpu.VMEM((bpw,), jnp.int32),
            pltpu.VMEM((bpw * K,), jnp.int32),
            pltpu.VMEM((NB, DPAD), jnp.float32),
            pltpu.VMEM((NB, DPAD), jnp.float32),
            pltpu.VMEM((NB * K, DPAD), jnp.float32),
            pltpu.VMEM((NB * K, DPAD), jnp.float32),
            pltpu.VMEM((NB, K), jnp.float32),
            pltpu.VMEM((NB, K), jnp.float32),
            pltpu.SemaphoreType.DMA,
            pltpu.SemaphoreType.DMA,
        ],
    )
    def k(table_hbm, widx_hbm, cidx_hbm, out_hbm,
          widx_v, cidx_v, wrows0, wrows1, crows0, crows1, out0, out1,
          sem0, sem1):
        wid = lax.axis_index("s") * NC + lax.axis_index("c")
        base_b = wid * bpw
        pltpu.sync_copy(widx_hbm.at[pl.ds(base_b, bpw)], widx_v)
        pltpu.sync_copy(cidx_hbm.at[pl.ds(base_b * K, bpw * K)], cidx_v)

        wrows = (wrows0, wrows1)
        crows = (crows0, crows1)
        outs = (out0, out1)
        sems = (sem0, sem1)
        iota = lax.iota(jnp.int32, 16)

        def fire(c, buf):
            woff = pl.multiple_of(c * NB, NB)
            coff = pl.multiple_of(c * (NB * K), NB * K)
            pltpu.async_copy(
                table_hbm.at[widx_v.at[pl.ds(woff, NB)]], wrows[buf],
                sems[buf])
            for j in range(nidx):
                pltpu.async_copy(
                    table_hbm.at[cidx_v.at[pl.ds(coff + j * IDXSTREAM,
                                                 IDXSTREAM)]],
                    crows[buf].at[pl.ds(j * IDXSTREAM, IDXSTREAM)], sems[buf])

        def drain(buf):
            pltpu.make_async_copy(
                table_hbm.at[pl.ds(0, NB)], wrows[buf], sems[buf]).wait()
            pltpu.make_async_copy(
                table_hbm.at[pl.ds(0, NB * K)], crows[buf], sems[buf]).wait()

        def compute(c, buf):
            crow_base = [iota * K + kk for kk in range(K)]

            def dbody(d, accs):
                dcol = (iota + d) & (D - 1)
                wcol = plsc.load_gather(wrows[buf], [iota, dcol])
                return tuple(
                    accs[kk] + wcol * plsc.load_gather(
                        crows[buf], [crow_base[kk], dcol])
                    for kk in range(K))

            accs = lax.fori_loop(
                0, D, dbody,
                tuple(jnp.zeros((L,), jnp.float32) for _ in range(K)),
                unroll=UNROLL)

            for kk in range(K):
                y = 1.0 / (1.0 + jnp.exp(-accs[kk]))
                plsc.store_scatter(
                    outs[buf], [iota, jnp.full((L,), kk, jnp.int32)], y)

            woff = pl.multiple_of(c * NB, NB)
            pltpu.sync_copy(outs[buf], out_hbm.at[pl.ds(base_b + woff, NB)])

        fire(0, 0)

        def pair_body(p, carry):
            c0 = p * 2
            fire(c0 + 1, 1)
            drain(0)
            compute(c0, 0)
            fire(jnp.minimum(c0 + 2, nchunk - 2), 0)
            drain(1)
            compute(c0 + 1, 1)
            return carry

        lax.fori_loop(0, nchunk // 2, pair_body, 0)
        drain(0)

    return k


_TBLK = 16384


def _transpose_pad(table):
    V, D = table.shape
    tab_t = table.T
    grid = pl.cdiv(V, _TBLK)
    eye = jnp.eye(D, DPAD, dtype=jnp.float32)

    def body(tt_ref, eye_ref, out_ref):
        out_ref[...] = jax.lax.dot_general(
            tt_ref[...], eye_ref[...], (((0,), (0,)), ((), ())),
            preferred_element_type=jnp.float32)

    return pl.pallas_call(
        body,
        grid=(grid,),
        in_specs=[
            pl.BlockSpec((D, _TBLK), lambda i: (0, i)),
            pl.BlockSpec((D, DPAD), lambda i: (0, 0)),
        ],
        out_specs=pl.BlockSpec((_TBLK, DPAD), lambda i: (i, 0)),
        out_shape=jax.ShapeDtypeStruct((V, DPAD), jnp.float32),
        compiler_params=pltpu.CompilerParams(
            dimension_semantics=("arbitrary",)),
    )(tab_t, eye)


def kernel(word_vector, context_vector, table):
    B, K = context_vector.shape
    V, D = table.shape
    widx = word_vector.reshape(-1)
    cidx = context_vector.reshape(-1)
    table_pad = _transpose_pad(table)
    k = _make_kernel(B, K, D, V)
    return k(table_pad, widx, cidx)


# speedup vs baseline: 13.6485x; 1.0141x over previous
import functools

import jax
import jax.numpy as jnp
from jax import lax
from jax.experimental import pallas as pl
from jax.experimental.pallas import tpu as pltpu
from jax.experimental.pallas import tpu_sc as plsc

NC = 2
NS = 16
L = 16
NW = NC * NS

NB = 16
IDXSTREAM = 80
UNROLL = 8
DPAD = 128


def _make_kernel(B, K, D, V):
    bpw = B // NW
    nchunk = bpw // NB
    nidx = (NB * K) // IDXSTREAM
    assert NB * K == nidx * IDXSTREAM and IDXSTREAM % 8 == 0
    assert D % L == 0 and nchunk % 2 == 0 and NB == L

    mesh = plsc.VectorSubcoreMesh(core_axis_name="c", subcore_axis_name="s")

    @functools.partial(
        pl.kernel,
        mesh=mesh,
        compiler_params=pltpu.CompilerParams(
            needs_layout_passes=False, use_tc_tiling_on_sc=True),
        out_type=jax.ShapeDtypeStruct((B, K), jnp.float32),
        scratch_types=[
            pltpu.VMEM((bpw,), jnp.int32),
            pltpu.VMEM((bpw * K,), jnp.int32),
            pltpu.VMEM((NB, DPAD), jnp.float32),
            pltpu.VMEM((NB, DPAD), jnp.float32),
            pltpu.VMEM((NB * K, DPAD), jnp.float32),
            pltpu.VMEM((NB * K, DPAD), jnp.float32),
            pltpu.VMEM((NB, K), jnp.float32),
            pltpu.VMEM((NB, K), jnp.float32),
            pltpu.SemaphoreType.DMA,
            pltpu.SemaphoreType.DMA,
        ],
    )
    def k(table_hbm, widx_hbm, cidx_hbm, out_hbm,
          widx_v, cidx_v, wrows0, wrows1, crows0, crows1, out0, out1,
          sem0, sem1):
        wid = lax.axis_index("s") * NC + lax.axis_index("c")
        base_b = wid * bpw
        pltpu.sync_copy(widx_hbm.at[pl.ds(base_b, bpw)], widx_v)
        pltpu.sync_copy(cidx_hbm.at[pl.ds(base_b * K, bpw * K)], cidx_v)

        wrows = (wrows0, wrows1)
        crows = (crows0, crows1)
        outs = (out0, out1)
        sems = (sem0, sem1)
        iota = lax.iota(jnp.int32, 16)

        def fire(c, buf):
            woff = pl.multiple_of(c * NB, NB)
            coff = pl.multiple_of(c * (NB * K), NB * K)
            pltpu.async_copy(
                table_hbm.at[widx_v.at[pl.ds(woff, NB)]], wrows[buf],
                sems[buf])
            for j in range(nidx):
                pltpu.async_copy(
                    table_hbm.at[cidx_v.at[pl.ds(coff + j * IDXSTREAM,
                                                 IDXSTREAM)]],
                    crows[buf].at[pl.ds(j * IDXSTREAM, IDXSTREAM)], sems[buf])

        def drain(buf):
            pltpu.make_async_copy(
                table_hbm.at[pl.ds(0, NB)], wrows[buf], sems[buf]).wait()
            pltpu.make_async_copy(
                table_hbm.at[pl.ds(0, NB * K)], crows[buf], sems[buf]).wait()

        def compute(c, buf):
            crow_base = [iota * K + kk for kk in range(K)]

            def dbody(d, accs):
                dcol = (iota + d) & (D - 1)
                wcol = plsc.load_gather(wrows[buf], [iota, dcol])
                return tuple(
                    accs[kk] + wcol * plsc.load_gather(
                        crows[buf], [crow_base[kk], dcol])
                    for kk in range(K))

            accs = lax.fori_loop(
                0, D, dbody,
                tuple(jnp.zeros((L,), jnp.float32) for _ in range(K)),
                unroll=UNROLL)

            for kk in range(K):
                y = 1.0 / (1.0 + jnp.exp(-accs[kk]))
                plsc.store_scatter(
                    outs[buf], [iota, jnp.full((L,), kk, jnp.int32)], y)

            woff = pl.multiple_of(c * NB, NB)
            pltpu.sync_copy(outs[buf], out_hbm.at[pl.ds(base_b + woff, NB)])

        fire(0, 0)

        def pair_body(p, carry):
            c0 = p * 2
            fire(c0 + 1, 1)
            drain(0)
            compute(c0, 0)
            fire(jnp.minimum(c0 + 2, nchunk - 2), 0)
            drain(1)
            compute(c0 + 1, 1)
            return carry

        lax.fori_loop(0, nchunk // 2, pair_body, 0)
        drain(0)

    return k


_TBLK = 32768


def _transpose_pad(table):
    V, D = table.shape
    tab_t = table.T
    grid = pl.cdiv(V, _TBLK)
    eye = jnp.eye(D, DPAD, dtype=jnp.float32)

    def body(tt_ref, eye_ref, out_ref):
        out_ref[...] = jax.lax.dot_general(
            tt_ref[...], eye_ref[...], (((0,), (0,)), ((), ())),
            preferred_element_type=jnp.float32)

    return pl.pallas_call(
        body,
        grid=(grid,),
        in_specs=[
            pl.BlockSpec((D, _TBLK), lambda i: (0, i)),
            pl.BlockSpec((D, DPAD), lambda i: (0, 0)),
        ],
        out_specs=pl.BlockSpec((_TBLK, DPAD), lambda i: (i, 0)),
        out_shape=jax.ShapeDtypeStruct((V, DPAD), jnp.float32),
        compiler_params=pltpu.CompilerParams(
            dimension_semantics=("arbitrary",)),
    )(tab_t, eye)


def kernel(word_vector, context_vector, table):
    B, K = context_vector.shape
    V, D = table.shape
    widx = word_vector.reshape(-1)
    cidx = context_vector.reshape(-1)
    table_pad = _transpose_pad(table)
    k = _make_kernel(B, K, D, V)
    return k(table_pad, widx, cidx)


# speedup vs baseline: 15.4951x; 1.1353x over previous
import functools

import jax
import jax.numpy as jnp
from jax import lax
from jax.experimental import pallas as pl
from jax.experimental.pallas import tpu as pltpu
from jax.experimental.pallas import tpu_sc as plsc

NC = 2
NS = 16
L = 16
NW = NC * NS

NB = 16
IDXSTREAM = 80
UNROLL = 8
DPAD = 128

_TBLK = 7936
_NBLK = 64


def _make_kernel(B, K, D, V, half, npack):
    bpw = B // NW
    nchunk = bpw // NB
    nidx = (NB * K) // IDXSTREAM
    assert NB * K == nidx * IDXSTREAM and IDXSTREAM % 8 == 0
    assert D % L == 0 and nchunk % 2 == 0 and NB == L

    mesh = plsc.VectorSubcoreMesh(core_axis_name="c", subcore_axis_name="s")

    @functools.partial(
        pl.kernel,
        mesh=mesh,
        compiler_params=pltpu.CompilerParams(
            needs_layout_passes=False, use_tc_tiling_on_sc=True),
        out_type=jax.ShapeDtypeStruct((B, K), jnp.float32),
        scratch_types=[
            pltpu.VMEM((bpw,), jnp.int32),
            pltpu.VMEM((bpw * K,), jnp.int32),
            pltpu.VMEM((bpw,), jnp.int32),
            pltpu.VMEM((bpw * K,), jnp.int32),
            pltpu.VMEM((NB, DPAD), jnp.float32),
            pltpu.VMEM((NB, DPAD), jnp.float32),
            pltpu.VMEM((NB * K, DPAD), jnp.float32),
            pltpu.VMEM((NB * K, DPAD), jnp.float32),
            pltpu.VMEM((NB, K), jnp.float32),
            pltpu.VMEM((NB, K), jnp.float32),
            pltpu.SemaphoreType.DMA,
            pltpu.SemaphoreType.DMA,
        ],
    )
    def k(table_hbm, widx_hbm, cidx_hbm, out_hbm,
          widx_v, cidx_v, wco_v, cco_v,
          wrows0, wrows1, crows0, crows1, out0, out1,
          sem0, sem1):
        wid = lax.axis_index("s") * NC + lax.axis_index("c")
        base_b = wid * bpw
        pltpu.sync_copy(widx_hbm.at[pl.ds(base_b, bpw)], widx_v)
        pltpu.sync_copy(cidx_hbm.at[pl.ds(base_b * K, bpw * K)], cidx_v)

        def remap_w(j, carry):
            v = widx_v[pl.ds(j * L, L)]
            hi = v >= half
            widx_v[pl.ds(j * L, L)] = jnp.where(hi, v - half, v)
            wco_v[pl.ds(j * L, L)] = jnp.where(hi, D, 0).astype(jnp.int32)
            return carry

        def remap_c(j, carry):
            v = cidx_v[pl.ds(j * L, L)]
            hi = v >= half
            cidx_v[pl.ds(j * L, L)] = jnp.where(hi, v - half, v)
            cco_v[pl.ds(j * L, L)] = jnp.where(hi, D, 0).astype(jnp.int32)
            return carry

        lax.fori_loop(0, bpw // L, remap_w, 0, unroll=4)
        lax.fori_loop(0, (bpw * K) // L, remap_c, 0, unroll=4)

        wrows = (wrows0, wrows1)
        crows = (crows0, crows1)
        outs = (out0, out1)
        sems = (sem0, sem1)
        iota = lax.iota(jnp.int32, 16)

        def fire(c, buf):
            woff = pl.multiple_of(c * NB, NB)
            coff = pl.multiple_of(c * (NB * K), NB * K)
            pltpu.async_copy(
                table_hbm.at[widx_v.at[pl.ds(woff, NB)]], wrows[buf],
                sems[buf])
            for j in range(nidx):
                pltpu.async_copy(
                    table_hbm.at[cidx_v.at[pl.ds(coff + j * IDXSTREAM,
                                                 IDXSTREAM)]],
                    crows[buf].at[pl.ds(j * IDXSTREAM, IDXSTREAM)], sems[buf])

        def drain(buf):
            pltpu.make_async_copy(
                table_hbm.at[pl.ds(0, NB)], wrows[buf], sems[buf]).wait()
            pltpu.make_async_copy(
                table_hbm.at[pl.ds(0, NB * K)], crows[buf], sems[buf]).wait()

        def compute(c, buf):
            woff = pl.multiple_of(c * NB, NB)
            coff = pl.multiple_of(c * (NB * K), NB * K)
            wco = wco_v[pl.ds(woff, L)]
            kh = K // 2
            for g in range(2):
                ks = list(range(g * kh, (g + 1) * kh))
                crow_base = [iota * K + kk for kk in ks]
                ccos = [
                    plsc.load_gather(cco_v, [coff + iota * K + kk])
                    for kk in ks
                ]

                def dbody(d, accs):
                    dsk = (iota + d) & (D - 1)
                    wcol = plsc.load_gather(wrows[buf], [iota, wco + dsk])
                    return tuple(
                        accs[i] + wcol * plsc.load_gather(
                            crows[buf], [crow_base[i], ccos[i] + dsk])
                        for i in range(kh))

                accs = lax.fori_loop(
                    0, D, dbody,
                    tuple(jnp.zeros((L,), jnp.float32) for _ in range(kh)),
                    unroll=UNROLL)

                for i, kk in enumerate(ks):
                    y = 1.0 / (1.0 + jnp.exp(-accs[i]))
                    plsc.store_scatter(
                        outs[buf], [iota, jnp.full((L,), kk, jnp.int32)], y)

            pltpu.sync_copy(outs[buf], out_hbm.at[pl.ds(base_b + woff, NB)])

        fire(0, 0)

        def pair_body(p, carry):
            c0 = p * 2
            fire(c0 + 1, 1)
            drain(0)
            compute(c0, 0)
            fire(jnp.minimum(c0 + 2, nchunk - 2), 0)
            drain(1)
            compute(c0 + 1, 1)
            return carry

        lax.fori_loop(0, nchunk // 2, pair_body, 0)
        drain(0)

    return k


def _transpose_pack(table, half):
    V, D = table.shape
    tab_t = table.T
    eye_l = jnp.eye(D, 2 * D, dtype=jnp.float32)
    eye_r = jnp.eye(D, 2 * D, k=D, dtype=jnp.float32)
    nblk_l = half // _TBLK

    def body(ta_ref, tb_ref, el_ref, er_ref, out_ref):
        ya = jax.lax.dot_general(
            ta_ref[...], el_ref[...], (((0,), (0,)), ((), ())),
            preferred_element_type=jnp.float32)
        yb = jax.lax.dot_general(
            tb_ref[...], er_ref[...], (((0,), (0,)), ((), ())),
            preferred_element_type=jnp.float32)
        out_ref[...] = ya + yb

    return pl.pallas_call(
        body,
        grid=(_NBLK,),
        in_specs=[
            pl.BlockSpec((D, _TBLK), lambda i: (0, i)),
            pl.BlockSpec((D, _TBLK), lambda i: (0, i + nblk_l)),
            pl.BlockSpec((D, 2 * D), lambda i: (0, 0)),
            pl.BlockSpec((D, 2 * D), lambda i: (0, 0)),
        ],
        out_specs=pl.BlockSpec((_TBLK, 2 * D), lambda i: (i, 0)),
        out_shape=jax.ShapeDtypeStruct((_NBLK * _TBLK, 2 * D), jnp.float32),
        compiler_params=pltpu.CompilerParams(
            dimension_semantics=("arbitrary",)),
    )(tab_t, tab_t, eye_l, eye_r)


def kernel(word_vector, context_vector, table):
    B, K = context_vector.shape
    V, D = table.shape
    half = _TBLK * (_NBLK - 1)
    assert half < V <= _TBLK * _NBLK + half
    widx = word_vector.reshape(-1)
    cidx = context_vector.reshape(-1)
    table_pack = _transpose_pack(table, half)
    k = _make_kernel(B, K, D, V, half, table_pack.shape[0])
    return k(table_pack, widx, cidx)
